# baseline (speedup 1.0000x reference)
import jax
import jax.numpy as jnp
import numpy as np
from jax.experimental import pallas as pl

B, T, RESO, HID, CDIM, NBLK = 8, 50000, 32, 32, 32, 5
S = RESO ** 3
PADDING = 0.1


def _pos_enc(p):
    freqs = (2.0 ** np.arange(10)) * np.pi
    p2 = 2.0 * p - 1.0
    out = []
    for f in freqs:
        out.append(jnp.sin(f * p2))
        out.append(jnp.cos(f * p2))
    return jnp.concatenate(out, axis=2)


def _resblock(x, w0, b0, w1, b1, ws):
    net = jax.nn.relu(x) @ w0 + b0
    dx = jax.nn.relu(net) @ w1 + b1
    return x @ ws + dx


def _pool_local(net, idx):
    def one(nb, ib):
        seg = jax.ops.segment_max(nb, ib, num_segments=S)
        cnt = jax.ops.segment_sum(jnp.ones((nb.shape[0],), nb.dtype), ib, num_segments=S)
        return jnp.where(cnt[:, None] > 0, seg, 0.0)
    seg = jax.vmap(one)(net, idx)
    return jnp.take_along_axis(seg, idx[:, :, None], axis=1)


def _scatter_mean_grid(c, idx):
    def one(cb, ib):
        s = jax.ops.segment_sum(cb, ib, num_segments=S)
        cnt = jax.ops.segment_sum(jnp.ones((cb.shape[0],), cb.dtype), ib, num_segments=S)
        return s / jnp.maximum(cnt, 1.0)[:, None]
    return jax.vmap(one)(c, idx)


def _bias_add_kernel(x_ref, b_ref, o_ref):
    o_ref[...] = x_ref[...] + b_ref[...]


def kernel(p, fc_pos_w, fc_pos_b, blk_fc0_w, blk_fc0_b, blk_fc1_w, blk_fc1_b, blk_sc_w, fc_c_w, fc_c_b):
    p_sg = jax.lax.stop_gradient(p)
    p_nor = jnp.clip(p_sg / (1.0 + PADDING + 1e-3) + 0.5, 0.0, 1.0 - 1e-3)
    x = jnp.clip((p_nor * RESO).astype(jnp.int32), 0, RESO - 1)
    idx = x[..., 0] + RESO * (x[..., 1] + RESO * x[..., 2])
    net = _pos_enc(p) @ fc_pos_w + fc_pos_b
    net = _resblock(net, blk_fc0_w[0], blk_fc0_b[0], blk_fc1_w[0], blk_fc1_b[0], blk_sc_w[0])
    for i in range(1, NBLK):
        pooled = _pool_local(net, idx)
        net = jnp.concatenate([net, pooled], axis=2)
        net = _resblock(net, blk_fc0_w[i], blk_fc0_b[i], blk_fc1_w[i], blk_fc1_b[i], blk_sc_w[i])
    c_nb = net @ fc_c_w
    c = pl.pallas_call(
        _bias_add_kernel,
        out_shape=jax.ShapeDtypeStruct(c_nb.shape, c_nb.dtype),
        grid=(B, 10),
        in_specs=[
            pl.BlockSpec((1, T // 10, CDIM), lambda i, j: (i, j, 0)),
            pl.BlockSpec((CDIM,), lambda i, j: (0,)),
        ],
        out_specs=pl.BlockSpec((1, T // 10, CDIM), lambda i, j: (i, j, 0)),
    )(c_nb, fc_c_b)
    mean = _scatter_mean_grid(c, idx)
    fea_grid = jnp.transpose(mean, (0, 2, 1)).reshape(p.shape[0], CDIM, RESO, RESO, RESO)
    return (fea_grid, net, c)


# speedup vs baseline: 1.5413x; 1.5413x over previous
import functools

import jax
import jax.numpy as jnp
import numpy as np
from jax import lax
from jax.experimental import pallas as pl
from jax.experimental.pallas import tpu as pltpu
from jax.experimental.pallas import tpu_sc as plsc

B, T, RESO, HID, CDIM, NBLK = 8, 50000, 32, 32, 32, 5
S = RESO ** 3
PADDING = 0.1
NROWS = B * T
TT = 5000
NTILES = NROWS // TT
NPAD = NROWS + TT
FREQS = [float((2.0 ** k) * np.pi) for k in range(10)]

NW = 32
CHW = 1568
WPAD = NW * CHW
GCAP = 54144
BCAP = 50176
SLOT_DUMP = 1024
PID_DUMP = NROWS
NSEG = 1040


def _relu(x):
    return jnp.maximum(x, 0.0)


def _stage0_body(p_ref, wpe_ref, bpe_ref, w0_ref, b0_ref, w1_ref, b1_ref,
                 ws_ref, net_ref, idx_ref):
    pb = p_ref[...]
    pn = jnp.clip(pb / (1.0 + PADDING + 1e-3) + 0.5, 0.0, 1.0 - 1e-3)
    xi = jnp.clip((pn * RESO).astype(jnp.int32), 0, RESO - 1)
    idx = xi[:, 0] + RESO * xi[:, 1] + (RESO * RESO) * xi[:, 2]
    idx_ref[0, 0, :] = idx
    p2 = 2.0 * pb - 1.0
    acc = jnp.zeros((TT, 2 * HID), jnp.float32) + bpe_ref[...]
    for k in range(10):
        ang = FREQS[k] * p2
        acc += jnp.dot(jnp.sin(ang), wpe_ref[6 * k:6 * k + 3, :],
                       preferred_element_type=jnp.float32)
        acc += jnp.dot(jnp.cos(ang), wpe_ref[6 * k + 3:6 * k + 6, :],
                       preferred_element_type=jnp.float32)
    h = jnp.dot(_relu(acc), w0_ref[...], preferred_element_type=jnp.float32) + b0_ref[...]
    dx = jnp.dot(_relu(h), w1_ref[...], preferred_element_type=jnp.float32) + b1_ref[...]
    net_ref[...] = jnp.dot(acc, ws_ref[...], preferred_element_type=jnp.float32) + dx


def _tc_stage0(p2d, wpe, bpe, w0, b0, w1, b1, ws):
    return pl.pallas_call(
        _stage0_body,
        grid=(NTILES,),
        in_specs=[
            pl.BlockSpec((TT, 3), lambda i: (i, 0)),
            pl.BlockSpec((60, 2 * HID), lambda i: (0, 0)),
            pl.BlockSpec((2 * HID,), lambda i: (0,)),
            pl.BlockSpec((2 * HID, HID), lambda i: (0, 0)),
            pl.BlockSpec((HID,), lambda i: (0,)),
            pl.BlockSpec((HID, HID), lambda i: (0, 0)),
            pl.BlockSpec((HID,), lambda i: (0,)),
            pl.BlockSpec((2 * HID, HID), lambda i: (0, 0)),
        ],
        out_specs=[
            pl.BlockSpec((TT, HID), lambda i: (i, 0)),
            pl.BlockSpec((1, 1, TT), lambda i: (i, 0, 0)),
        ],
        out_shape=[
            jax.ShapeDtypeStruct((NPAD, HID), jnp.float32),
            jax.ShapeDtypeStruct((NTILES, 1, TT), jnp.int32),
        ],
    )(p2d, wpe, bpe, w0, b0, w1, b1, ws)


def _round_body(last, net_ref, pool_ref, w0a_ref, w0b_ref, b0_ref, w1_ref,
                b1_ref, wsa_ref, wsb_ref, fcw_ref, fcb_ref, out_ref,
                c_ref=None):
    x1 = net_ref[...]
    x2 = pool_ref[...]
    h = (jnp.dot(_relu(x1), w0a_ref[...], preferred_element_type=jnp.float32)
         + jnp.dot(_relu(x2), w0b_ref[...], preferred_element_type=jnp.float32)
         + b0_ref[...])
    dx = jnp.dot(_relu(h), w1_ref[...], preferred_element_type=jnp.float32) + b1_ref[...]
    out = (jnp.dot(x1, wsa_ref[...], preferred_element_type=jnp.float32)
           + jnp.dot(x2, wsb_ref[...], preferred_element_type=jnp.float32)
           + dx)
    out_ref[...] = out
    if last:
        c_ref[...] = jnp.dot(out, fcw_ref[...], preferred_element_type=jnp.float32) + fcb_ref[...]


def _tc_round(net_pad, pool_pad, w0, b0, w1, b1, ws, fcw, fcb, last):
    n_out = 2 if last else 1
    return pl.pallas_call(
        functools.partial(_round_body, last),
        grid=(NTILES,),
        in_specs=[
            pl.BlockSpec((TT, HID), lambda i: (i, 0)),
            pl.BlockSpec((TT, HID), lambda i: (i, 0)),
            pl.BlockSpec((HID, HID), lambda i: (0, 0)),
            pl.BlockSpec((HID, HID), lambda i: (0, 0)),
            pl.BlockSpec((HID,), lambda i: (0,)),
            pl.BlockSpec((HID, HID), lambda i: (0, 0)),
            pl.BlockSpec((HID,), lambda i: (0,)),
            pl.BlockSpec((HID, HID), lambda i: (0, 0)),
            pl.BlockSpec((HID, HID), lambda i: (0, 0)),
            pl.BlockSpec((HID, CDIM), lambda i: (0, 0)),
            pl.BlockSpec((CDIM,), lambda i: (0,)),
        ],
        out_specs=[pl.BlockSpec((TT, HID), lambda i: (i, 0)),
                   pl.BlockSpec((TT, CDIM), lambda i: (i, 0))][:n_out],
        out_shape=[jax.ShapeDtypeStruct((NPAD, HID), jnp.float32),
                   jax.ShapeDtypeStruct((NPAD, CDIM), jnp.float32)][:n_out],
    )(net_pad, pool_pad, w0[:HID], w0[HID:], b0, w1, b1, ws[:HID], ws[HID:],
      fcw, fcb)


_MESH = plsc.VectorSubcoreMesh(core_axis_name="c", subcore_axis_name="s")


def _wid():
    return lax.axis_index("s") * 2 + lax.axis_index("c")


def _owner(v, b):
    return ((v & 31) + 7 * ((v >> 5) & 31) + 13 * (v >> 10) + 5 * b) & 31


def _sget(ref, *idxs):
    return plsc.load_gather(
        ref, [jnp.full((16,), i, jnp.int32) for i in idxs])[0]


def _sset(ref, pos, val, lane0):
    plsc.store_scatter(ref, [jnp.full((16,), pos, jnp.int32)],
                       jnp.full((16,), val), mask=lane0)


def _sc_bucketize(idx_pad):
    @functools.partial(
        pl.kernel,
        out_type=[
            jax.ShapeDtypeStruct((B * NW * CHW,), jnp.int32),
            jax.ShapeDtypeStruct((B * NW * CHW,), jnp.int32),
            jax.ShapeDtypeStruct((NW * B * NW,), jnp.int32),
        ],
        mesh=_MESH,
        compiler_params=pltpu.CompilerParams(needs_layout_passes=False, use_tc_tiling_on_sc=False),
        scratch_types=[
            pltpu.VMEM((CHW,), jnp.int32),
            pltpu.VMEM((CHW,), jnp.int32),
            pltpu.VMEM((CHW,), jnp.int32),
            pltpu.VMEM((NW,), jnp.int32),
            pltpu.SMEM((NW,), jnp.int32),
            pltpu.SMEM((NW,), jnp.int32),
        ],
    )
    def k(idx_hbm, lpid_hbm, lslot_hbm, cnts_hbm, idxbuf, lpid, lslot, cnt_vm, cntv, cur):
        w = _wid()
        n = jnp.minimum(CHW, T - w * CHW)
        lane0 = lax.iota(jnp.int32, 16) == 0

        def per_b(b, _):
            pltpu.sync_copy(idx_hbm.at[pl.ds(pl.multiple_of(b * WPAD + w * CHW, 8), CHW)], idxbuf)

            def zero(o, _):
                cntv[o] = 0
                return 0
            lax.fori_loop(0, NW, zero, 0)

            def count(i, _):
                o2 = _owner(_sget(idxbuf, i), b)
                cntv[o2] = cntv[o2] + 1
                return 0
            lax.fori_loop(0, n, count, 0)

            def prefix(o, off):
                cur[o] = off
                _sset(cnt_vm, o, cntv[o], lane0)
                return off + cntv[o]
            lax.fori_loop(0, NW, prefix, jnp.int32(0))

            gbase = b * T + w * CHW

            def place(i, _):
                v = _sget(idxbuf, i)
                o = _owner(v, b)
                pos = cur[o]
                cur[o] = pos + 1
                _sset(lpid, pos, gbase + i, lane0)
                _sset(lslot, pos, v >> 5, lane0)
                return 0
            lax.fori_loop(0, n, place, 0)

            pltpu.sync_copy(lpid, lpid_hbm.at[pl.ds(pl.multiple_of((b * NW + w) * CHW, 8), CHW)])
            pltpu.sync_copy(lslot, lslot_hbm.at[pl.ds(pl.multiple_of((b * NW + w) * CHW, 8), CHW)])
            pltpu.sync_copy(cnt_vm, cnts_hbm.at[pl.ds(pl.multiple_of((w * B + b) * NW, 8), NW)])
            return 0
        lax.fori_loop(0, B, per_b, 0)

    return k(idx_pad)


def _sc_compact(lpid_h, lslot_h, cnts_h):
    @functools.partial(
        pl.kernel,
        out_type=[
            jax.ShapeDtypeStruct((B * GCAP,), jnp.int32),
            jax.ShapeDtypeStruct((B * GCAP,), jnp.int32),
            jax.ShapeDtypeStruct((B * NW * 16,), jnp.int32),
        ],
        mesh=_MESH,
        compiler_params=pltpu.CompilerParams(needs_layout_passes=False, use_tc_tiling_on_sc=False),
        scratch_types=[
            pltpu.VMEM((CHW,), jnp.int32),
            pltpu.VMEM((CHW,), jnp.int32),
            pltpu.VMEM((NW * B * NW,), jnp.int32),
            pltpu.VMEM((BCAP,), jnp.int32),
            pltpu.VMEM((BCAP,), jnp.int32),
            pltpu.VMEM((16,), jnp.int32),
        ],
    )
    def k(lpid_hbm, lslot_hbm, cnts_hbm, gpid_hbm, gslot_hbm, meta_hbm,
          stage_pid, stage_slot, ctab, bpid, bslot, mbuf):
        o = _wid()
        lane = lax.iota(jnp.int32, 16)
        lane0 = lane == 0
        pltpu.sync_copy(cnts_hbm, ctab)

        def per_b(b, _):
            def base_loop(oo, acc):
                def wsum(w, a):
                    return a + _sget(ctab, (w * B + b) * NW + oo)
                t = lax.fori_loop(0, NW, wsum, jnp.int32(0))
                return acc + (((t + 127) >> 7) << 7)
            base = lax.fori_loop(0, o, base_loop, jnp.int32(0))

            def per_w(w, fill):
                pltpu.sync_copy(lpid_hbm.at[pl.ds(pl.multiple_of((b * NW + w) * CHW, 8), CHW)], stage_pid)
                pltpu.sync_copy(lslot_hbm.at[pl.ds(pl.multiple_of((b * NW + w) * CHW, 8), CHW)], stage_slot)

                def pref(oo, a):
                    return a + _sget(ctab, (w * B + b) * NW + oo)
                loff = lax.fori_loop(0, o, pref, jnp.int32(0))
                cnt = _sget(ctab, (w * B + b) * NW + o)

                def cp(i, f):
                    _sset(bpid, f, _sget(stage_pid, loff + i), lane0)
                    _sset(bslot, f, _sget(stage_slot, loff + i), lane0)
                    return f + 1
                return lax.fori_loop(0, cnt, cp, fill)
            tot = lax.fori_loop(0, NW, per_w, jnp.int32(0))

            tpad = ((tot + 127) >> 7) << 7

            def st(i, _):
                _sset(bpid, i, jnp.int32(PID_DUMP), lane0)
                _sset(bslot, i, jnp.int32(SLOT_DUMP), lane0)
                return 0
            lax.fori_loop(tot, tpad, st, 0)
            nch = tpad >> 7

            def out(kk, _):
                pltpu.sync_copy(bpid.at[pl.ds(pl.multiple_of(kk * 128, 8), 128)],
                                gpid_hbm.at[pl.ds(pl.multiple_of(b * GCAP + base + kk * 128, 8), 128)])
                pltpu.sync_copy(bslot.at[pl.ds(pl.multiple_of(kk * 128, 8), 128)],
                                gslot_hbm.at[pl.ds(pl.multiple_of(b * GCAP + base + kk * 128, 8), 128)])
                return 0
            lax.fori_loop(0, nch, out, 0)

            mbuf[...] = jnp.where(lane == 0, base,
                                  jnp.where(lane == 1, nch, 0))
            pltpu.sync_copy(mbuf, meta_hbm.at[pl.ds(pl.multiple_of((b * NW + o) * 16, 8), 16)])
            return 0
        lax.fori_loop(0, B, per_b, 0)

    return k(lpid_h, lslot_h, cnts_h)


def _sc_pool(gpid_h, gslot_h, meta_h, net_pad):
    @functools.partial(
        pl.kernel,
        out_type=jax.ShapeDtypeStruct((NPAD, HID), jnp.float32),
        mesh=_MESH,
        compiler_params=pltpu.CompilerParams(needs_layout_passes=False, use_tc_tiling_on_sc=False),
        scratch_types=[
            pltpu.VMEM((NSEG, HID), jnp.float32),
            pltpu.VMEM((128, HID), jnp.float32),
            pltpu.VMEM((128, HID), jnp.float32),
            pltpu.VMEM((128,), jnp.int32),
            pltpu.VMEM((128,), jnp.int32),
            pltpu.VMEM((16,), jnp.int32),
            pltpu.SemaphoreType.DMA,
        ],
    )
    def k(gpid_hbm, gslot_hbm, meta_hbm, net_hbm, pooled_hbm,
          seg, rows, orows, pb, sb, mbuf, sem):
        o = _wid()
        lane = lax.iota(jnp.int32, 16)
        laneh = lane + 16
        neg = jnp.full((16,), -3.0e38, jnp.float32)

        def per_b(b, _):
            pltpu.sync_copy(meta_hbm.at[pl.ds(pl.multiple_of((b * NW + o) * 16, 8), 16)], mbuf)
            base = _sget(mbuf, 0)
            nch = _sget(mbuf, 1)

            def init(i, _):
                iv = jnp.full((16,), i, jnp.int32)
                plsc.store_scatter(seg, [iv, lane], neg)
                plsc.store_scatter(seg, [iv, laneh], neg)
                return 0
            lax.fori_loop(0, NSEG, init, 0)

            def chunk(kk, _):
                pltpu.sync_copy(gpid_hbm.at[pl.ds(pl.multiple_of(b * GCAP + base + kk * 128, 8), 128)], pb)
                pltpu.sync_copy(gslot_hbm.at[pl.ds(pl.multiple_of(b * GCAP + base + kk * 128, 8), 128)], sb)
                pltpu.async_copy(net_hbm.at[pb], rows, sem).wait()

                def pt(j, _):
                    jv = jnp.full((16,), j, jnp.int32)
                    sv = plsc.load_gather(sb, [jv])
                    r0 = plsc.load_gather(rows, [jv, lane])
                    r1 = plsc.load_gather(rows, [jv, laneh])
                    s0 = plsc.load_gather(seg, [sv, lane])
                    s1 = plsc.load_gather(seg, [sv, laneh])
                    plsc.store_scatter(seg, [sv, lane], jnp.maximum(s0, r0))
                    plsc.store_scatter(seg, [sv, laneh], jnp.maximum(s1, r1))
                    return 0
                lax.fori_loop(0, 128, pt, 0)
                return 0
            lax.fori_loop(0, nch, chunk, 0)

            def chunk2(kk, _):
                pltpu.sync_copy(gpid_hbm.at[pl.ds(pl.multiple_of(b * GCAP + base + kk * 128, 8), 128)], pb)
                pltpu.sync_copy(gslot_hbm.at[pl.ds(pl.multiple_of(b * GCAP + base + kk * 128, 8), 128)], sb)

                def pt(j, _):
                    jv = jnp.full((16,), j, jnp.int32)
                    sv = plsc.load_gather(sb, [jv])
                    plsc.store_scatter(orows, [jv, lane],
                                       plsc.load_gather(seg, [sv, lane]))
                    plsc.store_scatter(orows, [jv, laneh],
                                       plsc.load_gather(seg, [sv, laneh]))
                    return 0
                lax.fori_loop(0, 128, pt, 0)
                pltpu.async_copy(orows, pooled_hbm.at[pb], sem).wait()
                return 0
            lax.fori_loop(0, nch, chunk2, 0)
            return 0
        lax.fori_loop(0, B, per_b, 0)

    return k(gpid_h, gslot_h, meta_h, net_pad)


def _sc_mean(gpid_h, gslot_h, meta_h, c_pad):
    @functools.partial(
        pl.kernel,
        out_type=jax.ShapeDtypeStruct((B * S, CDIM), jnp.float32),
        mesh=_MESH,
        compiler_params=pltpu.CompilerParams(needs_layout_passes=False, use_tc_tiling_on_sc=False),
        scratch_types=[
            pltpu.VMEM((NSEG, CDIM), jnp.float32),
            pltpu.VMEM((NSEG,), jnp.int32),
            pltpu.VMEM((128, CDIM), jnp.float32),
            pltpu.VMEM((128,), jnp.int32),
            pltpu.VMEM((128,), jnp.int32),
            pltpu.VMEM((128,), jnp.int32),
            pltpu.VMEM((16,), jnp.int32),
            pltpu.SemaphoreType.DMA,
        ],
    )
    def k(gpid_hbm, gslot_hbm, meta_hbm, c_hbm, mean_hbm,
          seg, cnt, rows, pb, sb, gix, mbuf, sem):
        o = _wid()
        lane = lax.iota(jnp.int32, 16)
        laneh = lane + 16
        lane0 = lane == 0
        zf = jnp.zeros((16,), jnp.float32)
        zi = jnp.zeros((16,), jnp.int32)

        def per_b(b, _):
            pltpu.sync_copy(meta_hbm.at[pl.ds(pl.multiple_of((b * NW + o) * 16, 8), 16)], mbuf)
            base = _sget(mbuf, 0)
            nch = _sget(mbuf, 1)

            def init(i, _):
                iv = jnp.full((16,), i, jnp.int32)
                plsc.store_scatter(seg, [iv, lane], zf)
                plsc.store_scatter(seg, [iv, laneh], zf)
                return 0
            lax.fori_loop(0, NSEG, init, 0)

            def initc(i, _):
                plsc.store_scatter(cnt, [i * 16 + lane], zi)
                return 0
            lax.fori_loop(0, NSEG // 16, initc, 0)

            def chunk(kk, _):
                pltpu.sync_copy(gpid_hbm.at[pl.ds(pl.multiple_of(b * GCAP + base + kk * 128, 8), 128)], pb)
                pltpu.sync_copy(gslot_hbm.at[pl.ds(pl.multiple_of(b * GCAP + base + kk * 128, 8), 128)], sb)
                pltpu.async_copy(c_hbm.at[pb], rows, sem).wait()

                def pt(j, _):
                    jv = jnp.full((16,), j, jnp.int32)
                    sv = plsc.load_gather(sb, [jv])
                    r0 = plsc.load_gather(rows, [jv, lane])
                    r1 = plsc.load_gather(rows, [jv, laneh])
                    s0 = plsc.load_gather(seg, [sv, lane])
                    s1 = plsc.load_gather(seg, [sv, laneh])
                    plsc.store_scatter(seg, [sv, lane], s0 + r0)
                    plsc.store_scatter(seg, [sv, laneh], s1 + r1)
                    cv = plsc.load_gather(cnt, [sv])
                    plsc.store_scatter(cnt, [sv], cv + 1, mask=lane0)
                    return 0
                lax.fori_loop(0, 128, pt, 0)
                return 0
            lax.fori_loop(0, nch, chunk, 0)

            def kk_loop(kk, _):
                def fin(s, _):
                    sv = jnp.full((16,), s, jnp.int32)
                    den = jnp.maximum(
                        plsc.load_gather(cnt, [sv]).astype(jnp.float32), 1.0)
                    m0 = plsc.load_gather(seg, [sv, lane]) / den
                    m1 = plsc.load_gather(seg, [sv, laneh]) / den
                    plsc.store_scatter(seg, [sv, lane], m0)
                    plsc.store_scatter(seg, [sv, laneh], m1)
                    y = s & 31
                    z = s >> 5
                    x = (o - 7 * y - 13 * z - 5 * b) & 31
                    _sset(gix, s & 127, b * S + x + (s << 5), lane0)
                    return 0
                lax.fori_loop(kk * 128, (kk + 1) * 128, fin, 0)
                pltpu.async_copy(seg.at[pl.ds(pl.multiple_of(kk * 128, 8), 128)],
                                 mean_hbm.at[gix], sem).wait()
                return 0
            lax.fori_loop(0, 8, kk_loop, 0)
            return 0
        lax.fori_loop(0, B, per_b, 0)

    return k(gpid_h, gslot_h, meta_h, c_pad)


def kernel(p, fc_pos_w, fc_pos_b, blk_fc0_w, blk_fc0_b, blk_fc1_w, blk_fc1_b,
           blk_sc_w, fc_c_w, fc_c_b):
    p2d = p.reshape(NROWS, 3)
    net_pad, idx3 = _tc_stage0(p2d, fc_pos_w, fc_pos_b, blk_fc0_w[0],
                               blk_fc0_b[0], blk_fc1_w[0], blk_fc1_b[0],
                               blk_sc_w[0])
    idx_pad = jnp.pad(idx3.reshape(B, T), ((0, 0), (0, WPAD - T))).reshape(B * WPAD)
    lpid, lslot, cnts = _sc_bucketize(idx_pad)
    gpid, gslot, meta = _sc_compact(lpid, lslot, cnts)
    for i in range(1, NBLK):
        pool_pad = _sc_pool(gpid, gslot, meta, net_pad)
        last = i == NBLK - 1
        outs = _tc_round(net_pad, pool_pad, blk_fc0_w[i], blk_fc0_b[i],
                         blk_fc1_w[i], blk_fc1_b[i], blk_sc_w[i],
                         fc_c_w, fc_c_b, last)
        net_pad = outs[0]
    c_pad = outs[1]
    mean2d = _sc_mean(gpid, gslot, meta, c_pad)
    net = net_pad[:NROWS].reshape(B, T, HID)
    c = c_pad[:NROWS].reshape(B, T, CDIM)
    fea_grid = jnp.transpose(mean2d.reshape(B, S, CDIM), (0, 2, 1)).reshape(
        B, CDIM, RESO, RESO, RESO)
    return (fea_grid, net, c)
